# baseline (speedup 1.0000x reference)
import functools

import jax
import jax.numpy as jnp
from jax import lax
from jax.experimental import pallas as pl
from jax.experimental.pallas import tpu as pltpu
from jax.experimental.pallas import tpu_sc as plsc

_NC = 2
_NS = 16
_NW = _NC * _NS

_BATCH = 16384
_N = 200
_NCLS = 8
_HORIZON = 60
_D = _N * _NCLS
_TAB_ELEMS = _NCLS * _D

_VECS = _TAB_ELEMS // 16
_VPW = _VECS // _NW

_BPW = _BATCH // _NW
_CHUNK = 64
_NCHUNK = _BPW // _CHUNK

_mesh = plsc.VectorSubcoreMesh(
    core_axis_name="c", subcore_axis_name="s", num_cores=_NC, num_subcores=_NS
)


@functools.partial(
    pl.kernel,
    out_type=jax.ShapeDtypeStruct((_NW, _VPW * 16), jnp.float32),
    mesh=_mesh,
    compiler_params=pltpu.CompilerParams(needs_layout_passes=False),
    scratch_types=[
        pltpu.MemorySpace.VMEM((_NCLS * _NCLS * _HORIZON,), jnp.float32),
        pltpu.MemorySpace.VMEM((_N,), jnp.int32),
        pltpu.MemorySpace.VMEM((_VPW * 16,), jnp.float32),
    ],
)
def _build_table(tp_hbm, fm_hbm, tab_hbm, tp_v, fm_v, buf_v):
    wid = lax.axis_index("s") * _NC + lax.axis_index("c")
    pltpu.sync_copy(tp_hbm, tp_v)
    pltpu.sync_copy(fm_hbm, fm_v)
    lanes = lax.iota(jnp.int32, 16)
    for j in range(_VPW):
        v = wid * _VPW + j
        f = (v % (_D // 16)) * 16 + lanes
        cls = v // (_D // 16)
        n = f >> 3
        c = f & 7
        hn = jnp.minimum(plsc.load_gather(fm_v, [n]), _HORIZON - 1)
        idx = cls * (_NCLS * _HORIZON) + c * _HORIZON + hn
        buf_v[pl.ds(j * 16, 16)] = plsc.load_gather(tp_v, [idx])
    pltpu.sync_copy(buf_v, tab_hbm.at[wid])


@functools.partial(
    pl.kernel,
    out_type=jax.ShapeDtypeStruct((_BATCH, _D), jnp.float32),
    mesh=_mesh,
    compiler_params=pltpu.CompilerParams(use_tc_tiling_on_sc=False),
    scratch_types=[
        pltpu.MemorySpace.VMEM((_BPW,), jnp.int32),
        pltpu.MemorySpace.VMEM((_CHUNK, _D), jnp.float32),
        pltpu.SemaphoreType.DMA,
    ],
)
def _gather_rows(tab_hbm, cc_hbm, out_hbm, idx_v, rows_v, sem):
    wid = lax.axis_index("s") * _NC + lax.axis_index("c")
    base = wid * _BPW
    pltpu.sync_copy(cc_hbm.at[pl.ds(base, _BPW)], idx_v)
    for ci in range(_NCHUNK):
        pltpu.async_copy(
            tab_hbm.at[idx_v.at[pl.ds(ci * _CHUNK, _CHUNK)]], rows_v, sem
        ).wait()
        pltpu.sync_copy(rows_v, out_hbm.at[pl.ds(base + ci * _CHUNK, _CHUNK)])


def kernel(current_class, future_minutes, transition_probs):
    cc = current_class.astype(jnp.int32)
    fm = future_minutes.astype(jnp.int32)
    tp_flat = transition_probs.reshape(-1)
    tab = _build_table(tp_flat, fm).reshape(_NCLS, _D)
    out = _gather_rows(tab, cc)
    return out.reshape(_BATCH, _N, _NCLS)
